# baseline (speedup 1.0000x reference)
import functools

import jax
import jax.numpy as jnp
from jax import lax
from jax.experimental import pallas as pl
from jax.experimental.pallas import tpu as pltpu
from jax.experimental.pallas import tpu_sc as plsc

N_F = 26
VOCAB = 100000
E = 32
B = 16384

NC = 2
NS = 16
NW = NC * NS
B_PER_W = B // NW
NB = 64
NCHUNK = B_PER_W // NB
ROWS = N_F * NB
G = 128
NG = ROWS // G


def _sc_body(idx_hbm, tab_hbm, out_hbm, idx_v, gath_v, trans_v, sem):
    c = lax.axis_index("c")
    s = lax.axis_index("s")
    wid = s * NC + c
    base = wid * B_PER_W
    e_lo = lax.iota(jnp.int32, 16) * ROWS
    e_hi = e_lo + 16 * ROWS

    def chunk_body(ci, _):
        pltpu.sync_copy(idx_hbm.at[wid, ci], idx_v)
        copies = []
        for g in range(NG):
            copies.append(
                pltpu.async_copy(tab_hbm.at[idx_v.at[g]], gath_v.at[g], sem))
        for cp in copies:
            cp.wait()

        def f_body(f, _):
            def b_body(b, _):
                r = f * NB + b
                lo = gath_v[r // G, r % G, pl.ds(0, 16)]
                hi = gath_v[r // G, r % G, pl.ds(16, 16)]
                offv = jnp.full((16,), b * N_F + f, jnp.int32)
                plsc.store_scatter(trans_v, [e_lo + offv], lo)
                plsc.store_scatter(trans_v, [e_hi + offv], hi)
                return 0
            return lax.fori_loop(0, NB, b_body, 0)
        lax.fori_loop(0, N_F, f_body, 0)

        out_off = (base + ci * NB) * N_F
        ocopies = []
        for e in range(E):
            ocopies.append(
                pltpu.async_copy(
                    trans_v.at[pl.ds(e * ROWS, ROWS)],
                    out_hbm.at[e, pl.ds(out_off, ROWS)],
                    sem))
        for cp in ocopies:
            cp.wait()
        return 0

    lax.fori_loop(0, NCHUNK, chunk_body, 0)


@jax.jit
def kernel(inputs, tables):
    offs = (jnp.arange(N_F, dtype=jnp.int32) * VOCAB)[None, None, None, :]
    idx = inputs.reshape(NW, NCHUNK, NB, N_F) + offs
    idx = idx.transpose(0, 1, 3, 2).reshape(NW, NCHUNK, NG, G)
    tab_flat = tables.reshape(N_F * VOCAB, E)

    mesh = plsc.VectorSubcoreMesh(core_axis_name="c", subcore_axis_name="s")
    run = pl.kernel(
        _sc_body,
        out_type=jax.ShapeDtypeStruct((E, B * N_F), jnp.float32),
        mesh=mesh,
        scratch_types=[
            pltpu.VMEM((NG, G), jnp.int32),
            pltpu.VMEM((NG, G, E), jnp.float32),
            pltpu.VMEM((E * ROWS,), jnp.float32),
            pltpu.SemaphoreType.DMA,
        ],
        compiler_params=pltpu.CompilerParams(
            needs_layout_passes=False,
            use_tc_tiling_on_sc=False,
        ),
    )
    return run(idx, tab_flat).reshape(E, B, N_F)


# speedup vs baseline: 1.0383x; 1.0383x over previous
import functools

import jax
import jax.numpy as jnp
from jax import lax
from jax.experimental import pallas as pl
from jax.experimental.pallas import tpu as pltpu
from jax.experimental.pallas import tpu_sc as plsc

N_F = 26
VOCAB = 100000
E = 32
B = 16384

NC = 2
NS = 16
NW = NC * NS
B_PER_W = B // NW
NB = 64
NCHUNK = B_PER_W // NB
ROWS = N_F * NB
G = 128
NG = ROWS // G


def _sc_body(idx_hbm, tab_hbm, out_hbm, idx_v, gath_v, trans_v, sem):
    c = lax.axis_index("c")
    s = lax.axis_index("s")
    wid = s * NC + c
    base = wid * B_PER_W
    e_lo = lax.iota(jnp.int32, 16) * ROWS
    e_hi = e_lo + 16 * ROWS

    def chunk_body(ci, _):
        pltpu.sync_copy(idx_hbm.at[wid, ci], idx_v)
        copies = []
        for g in range(NG):
            copies.append(
                pltpu.async_copy(tab_hbm.at[idx_v.at[g]],
                                 gath_v.at[pl.ds(g * G, G)], sem))
        for cp in copies:
            cp.wait()

        @plsc.parallel_loop(0, ROWS, 1, unroll=8)
        def _(r):
            b = r & (NB - 1)
            f = r >> 6
            lo = gath_v[r, pl.ds(0, 16)]
            hi = gath_v[r, pl.ds(16, 16)]
            offv = jnp.full((16,), b * N_F + f, jnp.int32)
            plsc.store_scatter(trans_v, [e_lo + offv], lo)
            plsc.store_scatter(trans_v, [e_hi + offv], hi)

        out_off = (base + ci * NB) * N_F
        ocopies = []
        for e in range(E):
            ocopies.append(
                pltpu.async_copy(
                    trans_v.at[pl.ds(e * ROWS, ROWS)],
                    out_hbm.at[e, pl.ds(out_off, ROWS)],
                    sem))
        for cp in ocopies:
            cp.wait()
        return 0

    lax.fori_loop(0, NCHUNK, chunk_body, 0)


@jax.jit
def kernel(inputs, tables):
    offs = (jnp.arange(N_F, dtype=jnp.int32) * VOCAB)[None, None, None, :]
    idx = inputs.reshape(NW, NCHUNK, NB, N_F) + offs
    idx = idx.transpose(0, 1, 3, 2).reshape(NW, NCHUNK, NG, G)
    tab_flat = tables.reshape(N_F * VOCAB, E)

    mesh = plsc.VectorSubcoreMesh(core_axis_name="c", subcore_axis_name="s")
    run = pl.kernel(
        _sc_body,
        out_type=jax.ShapeDtypeStruct((E, B * N_F), jnp.float32),
        mesh=mesh,
        scratch_types=[
            pltpu.VMEM((NG, G), jnp.int32),
            pltpu.VMEM((ROWS, E), jnp.float32),
            pltpu.VMEM((E * ROWS,), jnp.float32),
            pltpu.SemaphoreType.DMA,
        ],
        compiler_params=pltpu.CompilerParams(
            needs_layout_passes=False,
            use_tc_tiling_on_sc=False,
        ),
    )
    return run(idx, tab_flat).reshape(E, B, N_F)


# speedup vs baseline: 7.3598x; 7.0885x over previous
import jax
import jax.numpy as jnp
from jax import lax
from jax.experimental import pallas as pl
from jax.experimental.pallas import tpu as pltpu
from jax.experimental.pallas import tpu_sc as plsc

N_F = 26
VOCAB = 100000
E = 32
B = 16384

NC = 2
NS = 16
NW = NC * NS
PAIRS = N_F * E
PPW = PAIRS // NW
IC = 8192
NCH = B // IC


def _sc_body(idx_hbm, tab_hbm, out_hbm, idx_v, slab_v, out_v):
    c = lax.axis_index("c")
    s = lax.axis_index("s")
    wid = s * NC + c

    def pair_body(i, _):
        p = wid * PPW + i
        f = p >> 5
        e = p & (E - 1)
        pltpu.sync_copy(tab_hbm.at[f, e], slab_v)

        def chunk_body(ch, _):
            pltpu.sync_copy(idx_hbm.at[f, pl.ds(ch * IC, IC)], idx_v)
            base = ch * IC

            @plsc.parallel_loop(0, IC // 16, 1, unroll=8)
            def _(j):
                iv = idx_v[pl.ds(j * 16, 16)]
                out_v[pl.ds(base + j * 16, 16)] = plsc.load_gather(slab_v, [iv])

            return 0

        lax.fori_loop(0, NCH, chunk_body, 0)
        pltpu.sync_copy(out_v, out_hbm.at[f, e])
        return 0

    lax.fori_loop(0, PPW, pair_body, 0)


@jax.jit
def kernel(inputs, tables):
    idx_t = inputs.T
    tab_t = tables.transpose(0, 2, 1)

    mesh = plsc.VectorSubcoreMesh(core_axis_name="c", subcore_axis_name="s")
    run = pl.kernel(
        _sc_body,
        out_type=jax.ShapeDtypeStruct((N_F, E, B), jnp.float32),
        mesh=mesh,
        scratch_types=[
            pltpu.VMEM((IC,), jnp.int32),
            pltpu.VMEM((VOCAB,), jnp.float32),
            pltpu.VMEM((B,), jnp.float32),
        ],
        compiler_params=pltpu.CompilerParams(
            needs_layout_passes=False,
            use_tc_tiling_on_sc=True,
        ),
    )
    return run(idx_t, tab_t).transpose(1, 2, 0)


# speedup vs baseline: 9.3137x; 1.2655x over previous
import jax
import jax.numpy as jnp
from jax import lax
from jax.experimental import pallas as pl
from jax.experimental.pallas import tpu as pltpu
from jax.experimental.pallas import tpu_sc as plsc

N_F = 26
VOCAB = 100000
E = 32
B = 16384

NC = 2
NS = 16
NW = NC * NS
PAIRS = N_F * E
PPW = PAIRS // NW
OC = 4096
NOC = B // OC


def _sc_body(idx_hbm, tab_hbm, out_hbm, idx_v, slab_v, out_v, sem):
    c = lax.axis_index("c")
    s = lax.axis_index("s")
    wid = s * NC + c

    def pair_body(i, _):
        p = wid * PPW + i
        f = p >> 5
        e = p & (E - 1)
        @pl.when((i == 0) | (f != ((p - 1) >> 5)))
        def _():
            pltpu.sync_copy(idx_hbm.at[f], idx_v)

        pltpu.sync_copy(tab_hbm.at[f, e], slab_v)

        copies = []
        for ch in range(NOC):
            if ch >= 2:
                copies[ch - 2].wait()
            slot = ch % 2
            base = ch * OC

            @plsc.parallel_loop(0, OC // 16, 1, unroll=8)
            def _(j):
                iv = idx_v[pl.ds(base + j * 16, 16)]
                out_v[slot, pl.ds(j * 16, 16)] = plsc.load_gather(slab_v, [iv])

            copies.append(
                pltpu.async_copy(out_v.at[slot],
                                 out_hbm.at[f, e, pl.ds(base, OC)], sem))
        copies[NOC - 2].wait()
        copies[NOC - 1].wait()
        return 0

    lax.fori_loop(0, PPW, pair_body, 0)


@jax.jit
def kernel(inputs, tables):
    idx_t = inputs.T
    tab_t = tables.transpose(0, 2, 1)

    mesh = plsc.VectorSubcoreMesh(core_axis_name="c", subcore_axis_name="s")
    run = pl.kernel(
        _sc_body,
        out_type=jax.ShapeDtypeStruct((N_F, E, B), jnp.float32),
        mesh=mesh,
        scratch_types=[
            pltpu.VMEM((B,), jnp.int32),
            pltpu.VMEM((VOCAB,), jnp.float32),
            pltpu.VMEM((2, OC), jnp.float32),
            pltpu.SemaphoreType.DMA,
        ],
        compiler_params=pltpu.CompilerParams(
            needs_layout_passes=False,
            use_tc_tiling_on_sc=True,
        ),
    )
    return run(idx_t, tab_t).transpose(1, 2, 0)
